# baseline (speedup 1.0000x reference)
import functools

import jax
import jax.numpy as jnp
from jax import lax
from jax.experimental import pallas as pl
from jax.experimental.pallas import tpu as pltpu
from jax.experimental.pallas import tpu_sc as plsc

VOCAB = 1000
NGRAM = 4
DIM = 32
BATCH = 1024
EPS = 1e-23

VPAD = 1024
BBLK = 256
NROWS = BATCH * NGRAM + VPAD


def _softplus(x):
    return jnp.log(1.0 + jnp.exp(x))


def _log_softplus_eps(t):
    w = jnp.exp2(t * 1.4426950408889634)
    c1 = t <= -5.0
    arg = jnp.where(c1, w + EPS, 1.0 + w)
    L1 = jnp.log(arg)
    term_a = L1 + w * (w * (5.0 / 24.0) - 0.5)
    term_b = jnp.log(L1 + EPS)
    return jnp.where(c1, term_a, term_b)


ROWW = 128
CHUNK = 80


def _make_sc_gather():
    info = plsc.get_sparse_core_info()
    nw = info.num_cores * info.num_subcores
    rows_per_w = NROWS // nw
    nchunks = rows_per_w // CHUNK

    mesh = plsc.VectorSubcoreMesh(core_axis_name="c", subcore_axis_name="s")

    @functools.partial(
        pl.kernel,
        out_type=jax.ShapeDtypeStruct((NROWS, ROWW), jnp.float32),
        mesh=mesh,
        scratch_types=[
            pltpu.VMEM((rows_per_w,), jnp.int32),
            pltpu.VMEM((rows_per_w, ROWW), jnp.float32),
            pltpu.SemaphoreType.DMA,
        ],
    )
    def gather_k(table_hbm, idx_hbm, out_hbm, idx_v, rows_v, sem):
        wid = lax.axis_index("s") * info.num_cores + lax.axis_index("c")
        base = wid * rows_per_w
        pltpu.sync_copy(idx_hbm.at[pl.ds(base, rows_per_w)], idx_v)
        copies = [
            pltpu.async_copy(
                table_hbm.at[idx_v.at[pl.ds(c * CHUNK, CHUNK)]],
                rows_v.at[pl.ds(c * CHUNK, CHUNK)],
                sem,
            )
            for c in range(nchunks)
        ]
        for cp in copies:
            cp.wait()
        pltpu.sync_copy(rows_v, out_hbm.at[pl.ds(base, rows_per_w)])

    return gather_k


@functools.cache
def _sc_gather_fn():
    return _make_sc_gather()


def _prep_body(wbv_ref, ctx_ref, pmw_ref, pmb_ref, pdw_ref, pdb_ref,
               z1t_ref, Z1t_ref, z2_ref, Z2_ref):
    wbt = wbv_ref[...].T
    z1 = wbt[0:DIM, :]
    z1t_ref[...] = z1
    Z1t_ref[...] = z1 + _softplus(wbt[DIM:2 * DIM, :])

    zacc = None
    dacc = None
    for g in range(NGRAM):
        cm = ctx_ref[:, g, 0:DIM]
        cd = ctx_ref[:, g, DIM:2 * DIM]
        mn = cm * pmw_ref[g:g + 1, :] + pmb_ref[g:g + 1, :]
        dl = _softplus(cd * pdw_ref[g:g + 1, :] + pdb_ref[g:g + 1, :])
        zacc = mn if zacc is None else zacc + mn
        dacc = dl if dacc is None else dacc + dl
    z2 = zacc * (1.0 / NGRAM)
    d2 = dacc * (1.0 / NGRAM)
    z2_ref[...] = z2
    Z2_ref[...] = z2 + _softplus(d2)


def _score_body(z1t_ref, Z1t_ref, z2_ref, Z2_ref, bias_ref, out_ref):
    d = pl.program_id(1)
    z1r = z1t_ref[0]
    Z1r = Z1t_ref[0]
    z2blk = z2_ref[...]
    Z2blk = Z2_ref[...]
    lane = lax.broadcasted_iota(jnp.int32, (BBLK, DIM), 1)
    sel = lane == d
    z2c = jnp.sum(jnp.where(sel, z2blk, 0.0), axis=1, keepdims=True)
    Z2c = jnp.sum(jnp.where(sel, Z2blk, 0.0), axis=1, keepdims=True)

    t = jnp.minimum(Z1r, Z2c) - jnp.maximum(z1r, z2c)
    term = _log_softplus_eps(t)

    @pl.when(d == 0)
    def _():
        out_ref[...] = term

    @pl.when(d != 0)
    def _():
        out_ref[...] += term

    @pl.when(d == DIM - 1)
    def _():
        acc = out_ref[...] + bias_ref[...]
        vlane = lax.broadcasted_iota(jnp.int32, (BBLK, VPAD), 1)
        acc = jnp.where(vlane < VOCAB, acc, -1e30)
        m = jnp.max(acc, axis=1, keepdims=True)
        lse = jnp.log(jnp.sum(jnp.exp(acc - m), axis=1, keepdims=True)) + m
        out_ref[...] = acc - lse


def kernel(x, word_boxes, embedding_bias, pos_delta_w, pos_delta_b,
           pos_min_w, pos_min_b):
    table = word_boxes.reshape(VOCAB * NGRAM, 2 * DIM).astype(jnp.float32)
    table = jnp.pad(table, ((0, 0), (0, ROWW - 2 * DIM)))

    xflat = x.reshape(-1).astype(jnp.int32)
    vidx = (jnp.arange(VOCAB, dtype=jnp.int32) * NGRAM)
    pad = jnp.zeros((NROWS - BATCH * NGRAM - VOCAB,), jnp.int32)
    idx = jnp.concatenate([xflat, vidx, pad])

    rows = _sc_gather_fn()(table, idx)[:, :2 * DIM]
    ctx = rows[:BATCH * NGRAM].reshape(BATCH, NGRAM, 2 * DIM)
    wbv = rows[BATCH * NGRAM:]

    z1t, Z1t, z2, Z2 = pl.pallas_call(
        _prep_body,
        out_shape=[
            jax.ShapeDtypeStruct((DIM, VPAD), jnp.float32),
            jax.ShapeDtypeStruct((DIM, VPAD), jnp.float32),
            jax.ShapeDtypeStruct((BATCH, DIM), jnp.float32),
            jax.ShapeDtypeStruct((BATCH, DIM), jnp.float32),
        ],
    )(wbv, ctx, pos_min_w, pos_min_b, pos_delta_w, pos_delta_b)

    z1t3 = z1t.reshape(DIM, 1, VPAD)
    Z1t3 = Z1t.reshape(DIM, 1, VPAD)
    bias_row = jnp.pad(embedding_bias.reshape(1, VOCAB),
                       ((0, 0), (0, VPAD - VOCAB)))

    grid = (BATCH // BBLK, DIM)
    out = pl.pallas_call(
        _score_body,
        grid=grid,
        in_specs=[
            pl.BlockSpec((1, 1, VPAD), lambda i, d: (d, 0, 0)),
            pl.BlockSpec((1, 1, VPAD), lambda i, d: (d, 0, 0)),
            pl.BlockSpec((BBLK, DIM), lambda i, d: (i, 0)),
            pl.BlockSpec((BBLK, DIM), lambda i, d: (i, 0)),
            pl.BlockSpec((1, VPAD), lambda i, d: (0, 0)),
        ],
        out_specs=pl.BlockSpec((BBLK, VPAD), lambda i, d: (i, 0)),
        out_shape=jax.ShapeDtypeStruct((BATCH, VPAD), jnp.float32),
        compiler_params=pltpu.CompilerParams(
            dimension_semantics=("arbitrary", "arbitrary")),
    )(z1t3, Z1t3, z2, Z2, bias_row)

    return out[:, :VOCAB]


# speedup vs baseline: 1.2410x; 1.2410x over previous
import functools

import jax
import jax.numpy as jnp
from jax import lax
from jax.experimental import pallas as pl
from jax.experimental.pallas import tpu as pltpu
from jax.experimental.pallas import tpu_sc as plsc

VOCAB = 1000
NGRAM = 4
DIM = 32
BATCH = 1024
EPS = 1e-23

VPAD = 1024
BBLK = 1024
NROWS = BATCH * NGRAM + VPAD


def _softplus(x):
    return jnp.log(1.0 + jnp.exp(x))


def _log_softplus_eps(t):
    w = jnp.exp(t)
    sp_a = w * (1.0 - 0.5 * w)
    sp_b = jnp.log(1.0 + w)
    sp = jnp.where(t <= -5.0, sp_a, sp_b)
    return jnp.log(sp + EPS)


ROWW = 128
CHUNK = 80


def _make_sc_gather():
    info = plsc.get_sparse_core_info()
    nw = info.num_cores * info.num_subcores
    rows_per_w = NROWS // nw
    nchunks = rows_per_w // CHUNK

    mesh = plsc.VectorSubcoreMesh(core_axis_name="c", subcore_axis_name="s")

    @functools.partial(
        pl.kernel,
        out_type=jax.ShapeDtypeStruct((NROWS, ROWW), jnp.float32),
        mesh=mesh,
        scratch_types=[
            pltpu.VMEM((rows_per_w,), jnp.int32),
            pltpu.VMEM((rows_per_w, ROWW), jnp.float32),
            pltpu.SemaphoreType.DMA,
        ],
    )
    def gather_k(table_hbm, idx_hbm, out_hbm, idx_v, rows_v, sem):
        wid = lax.axis_index("s") * info.num_cores + lax.axis_index("c")
        base = wid * rows_per_w
        pltpu.sync_copy(idx_hbm.at[pl.ds(base, rows_per_w)], idx_v)
        copies = [
            pltpu.async_copy(
                table_hbm.at[idx_v.at[pl.ds(c * CHUNK, CHUNK)]],
                rows_v.at[pl.ds(c * CHUNK, CHUNK)],
                sem,
            )
            for c in range(nchunks)
        ]
        for cp in copies:
            cp.wait()
        pltpu.sync_copy(rows_v, out_hbm.at[pl.ds(base, rows_per_w)])

    return gather_k


@functools.cache
def _sc_gather_fn():
    return _make_sc_gather()


def _prep_body(wbv_ref, ctx_ref, pmw_ref, pmb_ref, pdw_ref, pdb_ref,
               z1t_ref, Z1t_ref, z2_ref, Z2_ref):
    wbt = wbv_ref[...].T
    z1 = wbt[0:DIM, :]
    z1t_ref[...] = z1
    Z1t_ref[...] = z1 + _softplus(wbt[DIM:2 * DIM, :])

    zacc = None
    dacc = None
    for g in range(NGRAM):
        cm = ctx_ref[:, g, 0:DIM]
        cd = ctx_ref[:, g, DIM:2 * DIM]
        mn = cm * pmw_ref[g:g + 1, :] + pmb_ref[g:g + 1, :]
        dl = _softplus(cd * pdw_ref[g:g + 1, :] + pdb_ref[g:g + 1, :])
        zacc = mn if zacc is None else zacc + mn
        dacc = dl if dacc is None else dacc + dl
    z2 = zacc * (1.0 / NGRAM)
    d2 = dacc * (1.0 / NGRAM)
    z2_ref[...] = z2
    Z2_ref[...] = z2 + _softplus(d2)


DPS = 2
NSTEP = DIM // DPS


def _score_body(z1t_ref, Z1t_ref, z2_ref, Z2_ref, bias_ref, out_ref):
    i = pl.program_id(0)
    z2blk = z2_ref[...]
    Z2blk = Z2_ref[...]
    lane = lax.broadcasted_iota(jnp.int32, (BBLK, DIM), 1)

    acc = None
    for k in range(DPS):
        d = i * DPS + k
        z1r = z1t_ref[d]
        Z1r = Z1t_ref[d]
        sel = lane == d
        z2c = jnp.sum(jnp.where(sel, z2blk, 0.0), axis=1, keepdims=True)
        Z2c = jnp.sum(jnp.where(sel, Z2blk, 0.0), axis=1, keepdims=True)
        t = jnp.minimum(Z1r, Z2c) - jnp.maximum(z1r, z2c)
        term = _log_softplus_eps(t)
        acc = term if acc is None else acc + term

    @pl.when(i == 0)
    def _():
        out_ref[...] = acc

    @pl.when(i != 0)
    def _():
        out_ref[...] += acc

    @pl.when(i == NSTEP - 1)
    def _():
        dec = out_ref[...] + bias_ref[...]
        vlane = lax.broadcasted_iota(jnp.int32, (BBLK, VPAD), 1)
        dec = jnp.where(vlane < VOCAB, dec, -1e30)
        m = jnp.max(dec, axis=1, keepdims=True)
        lse = jnp.log(jnp.sum(jnp.exp(dec - m), axis=1, keepdims=True)) + m
        out_ref[...] = dec - lse


def kernel(x, word_boxes, embedding_bias, pos_delta_w, pos_delta_b,
           pos_min_w, pos_min_b):
    table = word_boxes.reshape(VOCAB * NGRAM, 2 * DIM).astype(jnp.float32)
    table = jnp.pad(table, ((0, 0), (0, ROWW - 2 * DIM)))

    xflat = x.reshape(-1).astype(jnp.int32)
    vidx = (jnp.arange(VOCAB, dtype=jnp.int32) * NGRAM)
    pad = jnp.zeros((NROWS - BATCH * NGRAM - VOCAB,), jnp.int32)
    idx = jnp.concatenate([xflat, vidx, pad])

    rows = _sc_gather_fn()(table, idx)[:, :2 * DIM]
    ctx = rows[:BATCH * NGRAM].reshape(BATCH, NGRAM, 2 * DIM)
    wbv = rows[BATCH * NGRAM:]

    z1t, Z1t, z2, Z2 = pl.pallas_call(
        _prep_body,
        out_shape=[
            jax.ShapeDtypeStruct((DIM, VPAD), jnp.float32),
            jax.ShapeDtypeStruct((DIM, VPAD), jnp.float32),
            jax.ShapeDtypeStruct((BATCH, DIM), jnp.float32),
            jax.ShapeDtypeStruct((BATCH, DIM), jnp.float32),
        ],
    )(wbv, ctx, pos_min_w, pos_min_b, pos_delta_w, pos_delta_b)

    z1t3 = z1t.reshape(DIM, 1, VPAD)
    Z1t3 = Z1t.reshape(DIM, 1, VPAD)
    bias_row = jnp.pad(embedding_bias.reshape(1, VOCAB),
                       ((0, 0), (0, VPAD - VOCAB)))

    grid = (NSTEP,)
    out = pl.pallas_call(
        _score_body,
        grid=grid,
        in_specs=[
            pl.BlockSpec((DIM, 1, VPAD), lambda d: (0, 0, 0)),
            pl.BlockSpec((DIM, 1, VPAD), lambda d: (0, 0, 0)),
            pl.BlockSpec((BBLK, DIM), lambda d: (0, 0)),
            pl.BlockSpec((BBLK, DIM), lambda d: (0, 0)),
            pl.BlockSpec((1, VPAD), lambda d: (0, 0)),
        ],
        out_specs=pl.BlockSpec((BBLK, VPAD), lambda d: (0, 0)),
        out_shape=jax.ShapeDtypeStruct((BATCH, VPAD), jnp.float32),
        compiler_params=pltpu.CompilerParams(
            dimension_semantics=("arbitrary",)),
    )(z1t3, Z1t3, z2, Z2, bias_row)

    return out[:, :VOCAB]


# speedup vs baseline: 1.2759x; 1.0281x over previous
import functools

import jax
import jax.numpy as jnp
from jax import lax
from jax.experimental import pallas as pl
from jax.experimental.pallas import tpu as pltpu
from jax.experimental.pallas import tpu_sc as plsc

VOCAB = 1000
NGRAM = 4
DIM = 32
BATCH = 1024
EPS = 1e-23

VPAD = 1024
BBLK = 1024
NROWS = BATCH * NGRAM + VPAD


def _softplus(x):
    return jnp.log(1.0 + jnp.exp(x))


def _log_softplus_eps(t):
    w = jnp.exp(t)
    sp_a = w * (1.0 - 0.5 * w)
    sp_b = jnp.log(1.0 + w)
    sp = jnp.where(t <= -5.0, sp_a, sp_b)
    return jnp.log(sp + EPS)


ROWW = 128
CHUNK = 80


def _make_sc_gather():
    info = plsc.get_sparse_core_info()
    nw = info.num_cores * info.num_subcores
    rows_per_w = NROWS // nw
    nchunks = rows_per_w // CHUNK

    mesh = plsc.VectorSubcoreMesh(core_axis_name="c", subcore_axis_name="s")

    @functools.partial(
        pl.kernel,
        out_type=jax.ShapeDtypeStruct((NROWS, ROWW), jnp.float32),
        mesh=mesh,
        scratch_types=[
            pltpu.VMEM((rows_per_w,), jnp.int32),
            pltpu.VMEM((rows_per_w, ROWW), jnp.float32),
            pltpu.SemaphoreType.DMA,
        ],
    )
    def gather_k(table_hbm, idx_hbm, out_hbm, idx_v, rows_v, sem):
        wid = lax.axis_index("s") * info.num_cores + lax.axis_index("c")
        base = wid * rows_per_w
        pltpu.sync_copy(idx_hbm.at[pl.ds(base, rows_per_w)], idx_v)
        copies = [
            pltpu.async_copy(
                table_hbm.at[idx_v.at[pl.ds(c * CHUNK, CHUNK)]],
                rows_v.at[pl.ds(c * CHUNK, CHUNK)],
                sem,
            )
            for c in range(nchunks)
        ]
        for cp in copies:
            cp.wait()
        pltpu.sync_copy(rows_v, out_hbm.at[pl.ds(base, rows_per_w)])

    return gather_k


@functools.cache
def _sc_gather_fn():
    return _make_sc_gather()


def _prep_body(wbv_ref, ctx_ref, pmw_ref, pmb_ref, pdw_ref, pdb_ref,
               z1t_ref, Z1t_ref, z2_ref, Z2_ref):
    wbt = wbv_ref[...].T
    z1 = wbt[0:DIM, :]
    z1t_ref[...] = z1
    Z1t_ref[...] = z1 + _softplus(wbt[DIM:2 * DIM, :])

    zacc = None
    dacc = None
    for g in range(NGRAM):
        cm = ctx_ref[:, g, 0:DIM]
        cd = ctx_ref[:, g, DIM:2 * DIM]
        mn = cm * pmw_ref[g:g + 1, :] + pmb_ref[g:g + 1, :]
        dl = _softplus(cd * pdw_ref[g:g + 1, :] + pdb_ref[g:g + 1, :])
        zacc = mn if zacc is None else zacc + mn
        dacc = dl if dacc is None else dacc + dl
    z2 = zacc * (1.0 / NGRAM)
    d2 = dacc * (1.0 / NGRAM)
    z2_ref[...] = z2
    Z2_ref[...] = z2 + _softplus(d2)


DPS = 4
NSTEP = DIM // DPS


def _score_body(z1t_ref, Z1t_ref, z2_ref, Z2_ref, bias_ref, out_ref):
    i = pl.program_id(0)
    z2blk = z2_ref[...]
    Z2blk = Z2_ref[...]
    lane = lax.broadcasted_iota(jnp.int32, (BBLK, DIM), 1)

    acc = None
    for k in range(DPS):
        d = i * DPS + k
        z1r = z1t_ref[d]
        Z1r = Z1t_ref[d]
        sel = lane == d
        z2c = jnp.sum(jnp.where(sel, z2blk, 0.0), axis=1, keepdims=True)
        Z2c = jnp.sum(jnp.where(sel, Z2blk, 0.0), axis=1, keepdims=True)
        t = jnp.minimum(Z1r, Z2c) - jnp.maximum(z1r, z2c)
        term = _log_softplus_eps(t)
        acc = term if acc is None else acc + term

    @pl.when(i == 0)
    def _():
        out_ref[...] = acc

    @pl.when(i != 0)
    def _():
        out_ref[...] += acc

    @pl.when(i == NSTEP - 1)
    def _():
        dec = out_ref[...] + bias_ref[...]
        vlane = lax.broadcasted_iota(jnp.int32, (BBLK, VPAD), 1)
        dec = jnp.where(vlane < VOCAB, dec, -1e30)
        m = jnp.max(dec, axis=1, keepdims=True)
        lse = jnp.log(jnp.sum(jnp.exp(dec - m), axis=1, keepdims=True)) + m
        out_ref[...] = dec - lse


def kernel(x, word_boxes, embedding_bias, pos_delta_w, pos_delta_b,
           pos_min_w, pos_min_b):
    table = word_boxes.reshape(VOCAB * NGRAM, 2 * DIM).astype(jnp.float32)
    table = jnp.pad(table, ((0, 0), (0, ROWW - 2 * DIM)))

    xflat = x.reshape(-1).astype(jnp.int32)
    vidx = (jnp.arange(VOCAB, dtype=jnp.int32) * NGRAM)
    pad = jnp.zeros((NROWS - BATCH * NGRAM - VOCAB,), jnp.int32)
    idx = jnp.concatenate([xflat, vidx, pad])

    rows = _sc_gather_fn()(table, idx)[:, :2 * DIM]
    ctx = rows[:BATCH * NGRAM].reshape(BATCH, NGRAM, 2 * DIM)
    wbv = rows[BATCH * NGRAM:]

    z1t, Z1t, z2, Z2 = pl.pallas_call(
        _prep_body,
        out_shape=[
            jax.ShapeDtypeStruct((DIM, VPAD), jnp.float32),
            jax.ShapeDtypeStruct((DIM, VPAD), jnp.float32),
            jax.ShapeDtypeStruct((BATCH, DIM), jnp.float32),
            jax.ShapeDtypeStruct((BATCH, DIM), jnp.float32),
        ],
    )(wbv, ctx, pos_min_w, pos_min_b, pos_delta_w, pos_delta_b)

    z1t3 = z1t.reshape(DIM, 1, VPAD)
    Z1t3 = Z1t.reshape(DIM, 1, VPAD)
    bias_row = jnp.pad(embedding_bias.reshape(1, VOCAB),
                       ((0, 0), (0, VPAD - VOCAB)))

    grid = (NSTEP,)
    out = pl.pallas_call(
        _score_body,
        grid=grid,
        in_specs=[
            pl.BlockSpec((DIM, 1, VPAD), lambda d: (0, 0, 0)),
            pl.BlockSpec((DIM, 1, VPAD), lambda d: (0, 0, 0)),
            pl.BlockSpec((BBLK, DIM), lambda d: (0, 0)),
            pl.BlockSpec((BBLK, DIM), lambda d: (0, 0)),
            pl.BlockSpec((1, VPAD), lambda d: (0, 0)),
        ],
        out_specs=pl.BlockSpec((BBLK, VPAD), lambda d: (0, 0)),
        out_shape=jax.ShapeDtypeStruct((BATCH, VPAD), jnp.float32),
        compiler_params=pltpu.CompilerParams(
            dimension_semantics=("arbitrary",)),
    )(z1t3, Z1t3, z2, Z2, bias_row)

    return out[:, :VOCAB]


# speedup vs baseline: 1.3900x; 1.0894x over previous
import functools

import jax
import jax.numpy as jnp
from jax import lax
from jax.experimental import pallas as pl
from jax.experimental.pallas import tpu as pltpu
from jax.experimental.pallas import tpu_sc as plsc

VOCAB = 1000
NGRAM = 4
DIM = 32
BATCH = 1024
EPS = 1e-23

VPAD = 1024
BBLK = 1024
NCTX = BATCH * NGRAM

ROWW = 128
DPS = 4
NSTEP = DIM // DPS


def _softplus(x):
    return jnp.log(1.0 + jnp.exp(x))


def _sp_eps(t):
    w = jnp.exp(t)
    sp_a = w * (1.0 - 0.5 * w)
    sp_b = jnp.log(1.0 + w)
    return jnp.where(t <= -5.0, sp_a, sp_b) + EPS


def _make_sc_gather():
    info = plsc.get_sparse_core_info()
    nw = info.num_cores * info.num_subcores
    ctx_per_w = NCTX // nw
    voc_per_w = VPAD // nw

    mesh = plsc.VectorSubcoreMesh(core_axis_name="c", subcore_axis_name="s")

    @functools.partial(
        pl.kernel,
        out_type=[
            jax.ShapeDtypeStruct((NCTX, ROWW), jnp.float32),
            jax.ShapeDtypeStruct((VPAD, ROWW), jnp.float32),
        ],
        mesh=mesh,
        scratch_types=[
            pltpu.VMEM((ctx_per_w,), jnp.int32),
            pltpu.VMEM((voc_per_w,), jnp.int32),
            pltpu.VMEM((ctx_per_w, ROWW), jnp.float32),
            pltpu.VMEM((voc_per_w, ROWW), jnp.float32),
            pltpu.SemaphoreType.DMA,
        ],
    )
    def gather_k(table_hbm, xidx_hbm, vidx_hbm, ctx_hbm, wbv_hbm,
                 xi_v, vi_v, ctx_v, wbv_v, sem):
        wid = lax.axis_index("s") * info.num_cores + lax.axis_index("c")
        cbase = wid * ctx_per_w
        vbase = wid * voc_per_w
        pltpu.sync_copy(xidx_hbm.at[pl.ds(cbase, ctx_per_w)], xi_v)
        pltpu.sync_copy(vidx_hbm.at[pl.ds(vbase, voc_per_w)], vi_v)
        c1 = pltpu.async_copy(table_hbm.at[xi_v], ctx_v, sem)
        c2 = pltpu.async_copy(table_hbm.at[vi_v], wbv_v, sem)
        c1.wait()
        c2.wait()
        pltpu.sync_copy(ctx_v, ctx_hbm.at[pl.ds(cbase, ctx_per_w)])
        pltpu.sync_copy(wbv_v, wbv_hbm.at[pl.ds(vbase, voc_per_w)])

    return gather_k


@functools.cache
def _sc_gather_fn():
    return _make_sc_gather()


def _score_body(ctx_ref, wbv_ref, pmw_ref, pmb_ref, pdw_ref, pdb_ref,
                bias_ref, out_ref, z1t_s, Z1t_s, z2_s, Z2_s):
    i = pl.program_id(0)

    @pl.when(i == 0)
    def _prep():
        wbt = wbv_ref[:, 0:2 * DIM].T
        z1 = wbt[0:DIM, :]
        z1t_s[...] = z1.reshape(DIM, 1, VPAD)
        Z1t_s[...] = (z1 + _softplus(wbt[DIM:2 * DIM, :])).reshape(DIM, 1, VPAD)
        zacc = None
        dacc = None
        for g in range(NGRAM):
            cm = ctx_ref[:, g, 0:DIM]
            cd = ctx_ref[:, g, DIM:2 * DIM]
            mn = cm * pmw_ref[g:g + 1, :] + pmb_ref[g:g + 1, :]
            dl = _softplus(cd * pdw_ref[g:g + 1, :] + pdb_ref[g:g + 1, :])
            zacc = mn if zacc is None else zacc + mn
            dacc = dl if dacc is None else dacc + dl
        z2 = zacc * (1.0 / NGRAM)
        d2 = dacc * (1.0 / NGRAM)
        z2_s[...] = z2
        Z2_s[...] = z2 + _softplus(d2)

    z2blk = z2_s[...]
    Z2blk = Z2_s[...]
    lane = lax.broadcasted_iota(jnp.int32, (BBLK, DIM), 1)

    sps = []
    for k in range(DPS):
        d = i * DPS + k
        z1r = z1t_s[d]
        Z1r = Z1t_s[d]
        sel = lane == d
        z2c = jnp.sum(jnp.where(sel, z2blk, 0.0), axis=1, keepdims=True)
        Z2c = jnp.sum(jnp.where(sel, Z2blk, 0.0), axis=1, keepdims=True)
        t = jnp.minimum(Z1r, Z2c) - jnp.maximum(z1r, z2c)
        sps.append(_sp_eps(t))

    acc = None
    for k in range(0, DPS, 2):
        pair = jnp.log(jnp.maximum(sps[k] * sps[k + 1], 1.2e-38))
        acc = pair if acc is None else acc + pair

    @pl.when(i == 0)
    def _():
        out_ref[...] = acc

    @pl.when(i != 0)
    def _():
        out_ref[...] += acc

    @pl.when(i == NSTEP - 1)
    def _():
        dec = out_ref[...] + bias_ref[...]
        vlane = lax.broadcasted_iota(jnp.int32, (BBLK, VPAD), 1)
        dec = jnp.where(vlane < VOCAB, dec, -1e30)
        m = jnp.max(dec, axis=1, keepdims=True)
        lse = jnp.log(jnp.sum(jnp.exp(dec - m), axis=1, keepdims=True)) + m
        out_ref[...] = dec - lse


def kernel(x, word_boxes, embedding_bias, pos_delta_w, pos_delta_b,
           pos_min_w, pos_min_b):
    table = word_boxes.reshape(VOCAB * NGRAM, 2 * DIM).astype(jnp.float32)
    table = jnp.pad(table, ((0, 0), (0, ROWW - 2 * DIM)))

    xflat = x.reshape(-1).astype(jnp.int32)
    vidx = jnp.pad(jnp.arange(VOCAB, dtype=jnp.int32) * NGRAM,
                   (0, VPAD - VOCAB))

    ctx_rows, wbv_rows = _sc_gather_fn()(table, xflat, vidx)
    ctx4 = ctx_rows.reshape(BATCH, NGRAM, ROWW)

    bias_row = jnp.pad(embedding_bias.reshape(1, VOCAB),
                       ((0, 0), (0, VPAD - VOCAB)))

    out = pl.pallas_call(
        _score_body,
        grid=(NSTEP,),
        in_specs=[
            pl.BlockSpec((BATCH, NGRAM, ROWW), lambda d: (0, 0, 0)),
            pl.BlockSpec((VPAD, ROWW), lambda d: (0, 0)),
            pl.BlockSpec((NGRAM, DIM), lambda d: (0, 0)),
            pl.BlockSpec((NGRAM, DIM), lambda d: (0, 0)),
            pl.BlockSpec((NGRAM, DIM), lambda d: (0, 0)),
            pl.BlockSpec((NGRAM, DIM), lambda d: (0, 0)),
            pl.BlockSpec((1, VPAD), lambda d: (0, 0)),
        ],
        out_specs=pl.BlockSpec((BBLK, VPAD), lambda d: (0, 0)),
        out_shape=jax.ShapeDtypeStruct((BATCH, VPAD), jnp.float32),
        scratch_shapes=[
            pltpu.VMEM((DIM, 1, VPAD), jnp.float32),
            pltpu.VMEM((DIM, 1, VPAD), jnp.float32),
            pltpu.VMEM((BATCH, DIM), jnp.float32),
            pltpu.VMEM((BATCH, DIM), jnp.float32),
        ],
        compiler_params=pltpu.CompilerParams(
            dimension_semantics=("arbitrary",)),
    )(ctx4, wbv_rows, pos_min_w, pos_min_b, pos_delta_w, pos_delta_b,
      bias_row)

    return out[:, :VOCAB]


# speedup vs baseline: 1.4787x; 1.0638x over previous
import functools

import jax
import jax.numpy as jnp
from jax import lax
from jax.experimental import pallas as pl
from jax.experimental.pallas import tpu as pltpu
from jax.experimental.pallas import tpu_sc as plsc

VOCAB = 1000
NGRAM = 4
DIM = 32
BATCH = 1024
EPS = 1e-23

VPAD = 1024
BBLK = 1024
NCTX = BATCH * NGRAM

ROWW = 128
DPS = 4
NSTEP = DIM // DPS


def _softplus(x):
    return jnp.log(1.0 + jnp.exp(x))


def _sp_eps_from_w(w):
    sp_a = w * (1.0 - 0.5 * w)
    sp_b = jnp.log(1.0 + w)
    return jnp.where(w <= 6.7379470e-3, sp_a, sp_b) + EPS


def _make_sc_gather():
    info = plsc.get_sparse_core_info()
    nw = info.num_cores * info.num_subcores
    ctx_per_w = NCTX // nw
    voc_per_w = VPAD // nw

    mesh = plsc.VectorSubcoreMesh(core_axis_name="c", subcore_axis_name="s")

    @functools.partial(
        pl.kernel,
        out_type=[
            jax.ShapeDtypeStruct((NCTX, ROWW), jnp.float32),
            jax.ShapeDtypeStruct((VPAD, ROWW), jnp.float32),
        ],
        mesh=mesh,
        scratch_types=[
            pltpu.VMEM((ctx_per_w,), jnp.int32),
            pltpu.VMEM((voc_per_w,), jnp.int32),
            pltpu.VMEM((ctx_per_w, ROWW), jnp.float32),
            pltpu.VMEM((voc_per_w, ROWW), jnp.float32),
            pltpu.SemaphoreType.DMA,
        ],
    )
    def gather_k(table_hbm, xidx_hbm, vidx_hbm, ctx_hbm, wbv_hbm,
                 xi_v, vi_v, ctx_v, wbv_v, sem):
        wid = lax.axis_index("s") * info.num_cores + lax.axis_index("c")
        cbase = wid * ctx_per_w
        vbase = wid * voc_per_w
        pltpu.sync_copy(xidx_hbm.at[pl.ds(cbase, ctx_per_w)], xi_v)
        pltpu.sync_copy(vidx_hbm.at[pl.ds(vbase, voc_per_w)], vi_v)
        c1 = pltpu.async_copy(table_hbm.at[xi_v], ctx_v, sem)
        c2 = pltpu.async_copy(table_hbm.at[vi_v], wbv_v, sem)
        c1.wait()
        c2.wait()
        pltpu.sync_copy(ctx_v, ctx_hbm.at[pl.ds(cbase, ctx_per_w)])
        pltpu.sync_copy(wbv_v, wbv_hbm.at[pl.ds(vbase, voc_per_w)])

    return gather_k


@functools.cache
def _sc_gather_fn():
    return _make_sc_gather()


def _score_body(ctx_ref, wbv_ref, pmw_ref, pmb_ref, pdw_ref, pdb_ref,
                bias_ref, out_ref, E1t_s, Ei1t_s, E2_s, Ei2_s):
    i = pl.program_id(0)

    @pl.when(i == 0)
    def _prep():
        wbt = wbv_ref[:, 0:2 * DIM].T
        z1 = wbt[0:DIM, :]
        Z1 = z1 + _softplus(wbt[DIM:2 * DIM, :])
        E1t_s[...] = jnp.exp(Z1).reshape(DIM, 1, VPAD)
        Ei1t_s[...] = jnp.exp(-z1).reshape(DIM, 1, VPAD)
        zacc = None
        dacc = None
        for g in range(NGRAM):
            cm = ctx_ref[:, g, 0:DIM]
            cd = ctx_ref[:, g, DIM:2 * DIM]
            mn = cm * pmw_ref[g:g + 1, :] + pmb_ref[g:g + 1, :]
            dl = _softplus(cd * pdw_ref[g:g + 1, :] + pdb_ref[g:g + 1, :])
            zacc = mn if zacc is None else zacc + mn
            dacc = dl if dacc is None else dacc + dl
        z2 = zacc * (1.0 / NGRAM)
        d2 = dacc * (1.0 / NGRAM)
        E2_s[...] = jnp.exp(z2 + _softplus(d2))
        Ei2_s[...] = jnp.exp(-z2)

    E2blk = E2_s[...]
    Ei2blk = Ei2_s[...]
    lane = lax.broadcasted_iota(jnp.int32, (BBLK, DIM), 1)

    sps = []
    for k in range(DPS):
        d = i * DPS + k
        E1r = E1t_s[d]
        Ei1r = Ei1t_s[d]
        sel = lane == d
        E2c = jnp.sum(jnp.where(sel, E2blk, 0.0), axis=1, keepdims=True)
        Ei2c = jnp.sum(jnp.where(sel, Ei2blk, 0.0), axis=1, keepdims=True)
        w = jnp.minimum(E1r, E2c) * jnp.minimum(Ei1r, Ei2c)
        sps.append(_sp_eps_from_w(w))

    acc = None
    for k in range(0, DPS, 2):
        pair = jnp.log(jnp.maximum(sps[k] * sps[k + 1], 1.2e-38))
        acc = pair if acc is None else acc + pair

    @pl.when(i == 0)
    def _():
        out_ref[...] = acc

    @pl.when(i != 0)
    def _():
        out_ref[...] += acc

    @pl.when(i == NSTEP - 1)
    def _():
        dec = out_ref[...] + bias_ref[...]
        vlane = lax.broadcasted_iota(jnp.int32, (BBLK, VPAD), 1)
        dec = jnp.where(vlane < VOCAB, dec, -1e30)
        m = jnp.max(dec, axis=1, keepdims=True)
        lse = jnp.log(jnp.sum(jnp.exp(dec - m), axis=1, keepdims=True)) + m
        out_ref[...] = dec - lse


def kernel(x, word_boxes, embedding_bias, pos_delta_w, pos_delta_b,
           pos_min_w, pos_min_b):
    table = word_boxes.reshape(VOCAB * NGRAM, 2 * DIM).astype(jnp.float32)
    table = jnp.pad(table, ((0, 0), (0, ROWW - 2 * DIM)))

    xflat = x.reshape(-1).astype(jnp.int32)
    vidx = jnp.pad(jnp.arange(VOCAB, dtype=jnp.int32) * NGRAM,
                   (0, VPAD - VOCAB))

    ctx_rows, wbv_rows = _sc_gather_fn()(table, xflat, vidx)
    ctx4 = ctx_rows.reshape(BATCH, NGRAM, ROWW)

    bias_row = jnp.pad(embedding_bias.reshape(1, VOCAB),
                       ((0, 0), (0, VPAD - VOCAB)))

    out = pl.pallas_call(
        _score_body,
        grid=(NSTEP,),
        in_specs=[
            pl.BlockSpec((BATCH, NGRAM, ROWW), lambda d: (0, 0, 0)),
            pl.BlockSpec((VPAD, ROWW), lambda d: (0, 0)),
            pl.BlockSpec((NGRAM, DIM), lambda d: (0, 0)),
            pl.BlockSpec((NGRAM, DIM), lambda d: (0, 0)),
            pl.BlockSpec((NGRAM, DIM), lambda d: (0, 0)),
            pl.BlockSpec((NGRAM, DIM), lambda d: (0, 0)),
            pl.BlockSpec((1, VPAD), lambda d: (0, 0)),
        ],
        out_specs=pl.BlockSpec((BBLK, VPAD), lambda d: (0, 0)),
        out_shape=jax.ShapeDtypeStruct((BATCH, VPAD), jnp.float32),
        scratch_shapes=[
            pltpu.VMEM((DIM, 1, VPAD), jnp.float32),
            pltpu.VMEM((DIM, 1, VPAD), jnp.float32),
            pltpu.VMEM((BATCH, DIM), jnp.float32),
            pltpu.VMEM((BATCH, DIM), jnp.float32),
        ],
        compiler_params=pltpu.CompilerParams(
            dimension_semantics=("arbitrary",)),
    )(ctx4, wbv_rows, pos_min_w, pos_min_b, pos_delta_w, pos_delta_b,
      bias_row)

    return out[:, :VOCAB]
